# Initial kernel scaffold; baseline (speedup 1.0000x reference)
import jax
import jax.numpy as jnp
from jax.experimental import pallas as pl
from jax.experimental.pallas import tpu as pltpu

K = 3
R = 2
N = 512
KT = 3

_PREC = jax.lax.Precision.HIGHEST


def _dot(a, b):
    return jax.lax.dot_general(
        a, b, (((1,), (0,)), ((), ())),
        precision=_PREC, preferred_element_type=jnp.float32)


def _attn_stage_kernel(x0_ref, x1_ref, x2_ref, sup_ref, att_ref, w1_ref,
                       wt_ref, wl_ref, wr_ref, out_ref):
    w1 = w1_ref[...]
    y = _dot(x0_ref[0, 0], w1[0])
    y = y + _dot(x1_ref[0, 0], w1[1])
    y = y + _dot(x2_ref[0, 0], w1[2])
    ch = y.shape[-1] // 2
    h = y[:, :ch] * jax.nn.sigmoid(y[:, ch:])

    wl = wl_ref[...]
    wr = wr_ref[...]
    out = jnp.zeros_like(h)
    for k in range(K):
        wx = _dot(h, wt_ref[k])
        al = _dot(wx, wl[k].T)
        ar = jax.lax.dot_general(
            wr[k], wx, (((1,), (1,)), ((), ())),
            precision=_PREC, preferred_element_type=jnp.float32)
        s = jnp.zeros((N, N), dtype=jnp.float32)
        for i in range(R):
            m = (att_ref[i] != 0).astype(jnp.float32)
            s = s + (al[:, i:i + 1] + ar[i:i + 1, :]) * m
        m = (sup_ref[k] != 0).astype(jnp.float32)
        s = s + (al[:, R:R + 1] + ar[R:R + 1, :]) * m
        s = jnp.where(s == 0.0, -jnp.inf, s)
        mx = jnp.max(s, axis=-1, keepdims=True)
        e = jnp.exp(s - mx)
        z = jnp.sum(e, axis=-1, keepdims=True)
        p = e / z
        p = jnp.where(jnp.isnan(p), 0.0, p)
        out = out + _dot(p, wx)
    out_ref[0, 0] = jax.nn.elu(out)


def _conv2_ln_kernel(h0_ref, h1_ref, h2_ref, w2_ref, g_ref, b_ref, out_ref):
    w2 = w2_ref[...]
    y = _dot(h0_ref[0, 0], w2[0])
    y = y + _dot(h1_ref[0, 0], w2[1])
    y = y + _dot(h2_ref[0, 0], w2[2])
    ch = y.shape[-1] // 2
    g = y[:, :ch] * jax.nn.sigmoid(y[:, ch:])
    mu = jnp.mean(g)
    var = jnp.mean((g - mu) * (g - mu))
    out_ref[0, 0] = ((g - mu) / jnp.sqrt(var + 1e-6)) * g_ref[0, 0] + b_ref[0, 0]


def kernel(x, supports, atten_supports, W1, W_transform, W_left, W_right,
           W2, gamma, beta):
    B, T, n, cin = x.shape
    ch2 = W1.shape[-1]
    cs = W_transform.shape[-1]
    cout2 = W2.shape[-1]
    T1 = T - KT + 1
    T2 = T1 - KT + 1

    h2 = pl.pallas_call(
        _attn_stage_kernel,
        grid=(B, T1),
        in_specs=[
            pl.BlockSpec((1, 1, n, cin), lambda b, t: (b, t, 0, 0)),
            pl.BlockSpec((1, 1, n, cin), lambda b, t: (b, t + 1, 0, 0)),
            pl.BlockSpec((1, 1, n, cin), lambda b, t: (b, t + 2, 0, 0)),
            pl.BlockSpec((K, n, n), lambda b, t: (0, 0, 0)),
            pl.BlockSpec((R, n, n), lambda b, t: (0, 0, 0)),
            pl.BlockSpec((KT, cin, ch2), lambda b, t: (0, 0, 0)),
            pl.BlockSpec((K, ch2 // 2, cs), lambda b, t: (0, 0, 0)),
            pl.BlockSpec((K, R + 1, cs), lambda b, t: (0, 0, 0)),
            pl.BlockSpec((K, R + 1, cs), lambda b, t: (0, 0, 0)),
        ],
        out_specs=pl.BlockSpec((1, 1, n, cs), lambda b, t: (b, t, 0, 0)),
        out_shape=jax.ShapeDtypeStruct((B, T1, n, cs), jnp.float32),
        compiler_params=pltpu.CompilerParams(
            dimension_semantics=("parallel", "arbitrary")),
    )(x, x, x, supports, atten_supports, W1, W_transform, W_left, W_right)

    out = pl.pallas_call(
        _conv2_ln_kernel,
        grid=(B, T2),
        in_specs=[
            pl.BlockSpec((1, 1, n, cs), lambda b, t: (b, t, 0, 0)),
            pl.BlockSpec((1, 1, n, cs), lambda b, t: (b, t + 1, 0, 0)),
            pl.BlockSpec((1, 1, n, cs), lambda b, t: (b, t + 2, 0, 0)),
            pl.BlockSpec((KT, cs, cout2), lambda b, t: (0, 0, 0)),
            pl.BlockSpec((1, 1, n, cout2 // 2), lambda b, t: (0, 0, 0, 0)),
            pl.BlockSpec((1, 1, n, cout2 // 2), lambda b, t: (0, 0, 0, 0)),
        ],
        out_specs=pl.BlockSpec((1, 1, n, cout2 // 2), lambda b, t: (b, t, 0, 0)),
        out_shape=jax.ShapeDtypeStruct((B, T2, n, cout2 // 2), jnp.float32),
        compiler_params=pltpu.CompilerParams(
            dimension_semantics=("parallel", "arbitrary")),
    )(h2, h2, h2, W2, gamma, beta)
    return out


# speedup vs baseline: 1.6613x; 1.6613x over previous
import jax
import jax.numpy as jnp
from jax.experimental import pallas as pl
from jax.experimental.pallas import tpu as pltpu

K = 3
R = 2
N = 512
KT = 3

_PREC = jax.lax.Precision.HIGHEST


def _dot(a, b):
    return jax.lax.dot_general(
        a, b, (((1,), (0,)), ((), ())),
        precision=_PREC, preferred_element_type=jnp.float32)


def _attn_stage_kernel(x0_ref, x1_ref, x2_ref, sup_ref, att_ref, w1_ref,
                       wt_ref, wl_ref, wr_ref, out_ref):
    w1 = w1_ref[...]
    y = _dot(x0_ref[0, 0], w1[0])
    y = y + _dot(x1_ref[0, 0], w1[1])
    y = y + _dot(x2_ref[0, 0], w1[2])
    ch = y.shape[-1] // 2
    h = y[:, :ch] * jax.nn.sigmoid(y[:, ch:])

    wl = wl_ref[...]
    wr = wr_ref[...]
    out = jnp.zeros_like(h)
    for k in range(K):
        wx = _dot(h, wt_ref[k])
        al = _dot(wx, wl[k].T)
        ar = jax.lax.dot_general(
            wr[k], wx, (((1,), (1,)), ((), ())),
            precision=_PREC, preferred_element_type=jnp.float32)
        s = jnp.zeros((N, N), dtype=jnp.float32)
        for i in range(R):
            m = (att_ref[i] != 0).astype(jnp.float32)
            s = s + (al[:, i:i + 1] + ar[i:i + 1, :]) * m
        m = (sup_ref[k] != 0).astype(jnp.float32)
        s = s + (al[:, R:R + 1] + ar[R:R + 1, :]) * m
        s = jnp.where(s == 0.0, -jnp.inf, s)
        mx = jnp.max(s, axis=-1, keepdims=True)
        e = jnp.exp(s - mx)
        z = jnp.sum(e, axis=-1, keepdims=True)
        p = e / z
        p = jnp.where(jnp.isnan(p), 0.0, p)
        out = out + _dot(p, wx)
    out_ref[0, 0] = jnp.where(out > 0, out, jnp.exp(jnp.minimum(out, 0.0)) - 1.0)


def _conv2_ln_kernel(h0_ref, h1_ref, h2_ref, w2_ref, g_ref, b_ref, out_ref):
    w2 = w2_ref[...]
    y = _dot(h0_ref[0, 0], w2[0])
    y = y + _dot(h1_ref[0, 0], w2[1])
    y = y + _dot(h2_ref[0, 0], w2[2])
    ch = y.shape[-1] // 2
    g = y[:, :ch] * jax.nn.sigmoid(y[:, ch:])
    mu = jnp.mean(g)
    var = jnp.mean((g - mu) * (g - mu))
    out_ref[0, 0] = ((g - mu) / jnp.sqrt(var + 1e-6)) * g_ref[0, 0] + b_ref[0, 0]


def kernel(x, supports, atten_supports, W1, W_transform, W_left, W_right,
           W2, gamma, beta):
    B, T, n, cin = x.shape
    ch2 = W1.shape[-1]
    cs = W_transform.shape[-1]
    cout2 = W2.shape[-1]
    T1 = T - KT + 1
    T2 = T1 - KT + 1

    h2 = pl.pallas_call(
        _attn_stage_kernel,
        grid=(B, T1),
        in_specs=[
            pl.BlockSpec((1, 1, n, cin), lambda b, t: (b, t, 0, 0)),
            pl.BlockSpec((1, 1, n, cin), lambda b, t: (b, t + 1, 0, 0)),
            pl.BlockSpec((1, 1, n, cin), lambda b, t: (b, t + 2, 0, 0)),
            pl.BlockSpec((K, n, n), lambda b, t: (0, 0, 0)),
            pl.BlockSpec((R, n, n), lambda b, t: (0, 0, 0)),
            pl.BlockSpec((KT, cin, ch2), lambda b, t: (0, 0, 0)),
            pl.BlockSpec((K, ch2 // 2, cs), lambda b, t: (0, 0, 0)),
            pl.BlockSpec((K, R + 1, cs), lambda b, t: (0, 0, 0)),
            pl.BlockSpec((K, R + 1, cs), lambda b, t: (0, 0, 0)),
        ],
        out_specs=pl.BlockSpec((1, 1, n, cs), lambda b, t: (b, t, 0, 0)),
        out_shape=jax.ShapeDtypeStruct((B, T1, n, cs), jnp.float32),
        compiler_params=pltpu.CompilerParams(
            dimension_semantics=("parallel", "arbitrary")),
    )(x, x, x, supports, atten_supports, W1, W_transform, W_left, W_right)

    out = pl.pallas_call(
        _conv2_ln_kernel,
        grid=(B, T2),
        in_specs=[
            pl.BlockSpec((1, 1, n, cs), lambda b, t: (b, t, 0, 0)),
            pl.BlockSpec((1, 1, n, cs), lambda b, t: (b, t + 1, 0, 0)),
            pl.BlockSpec((1, 1, n, cs), lambda b, t: (b, t + 2, 0, 0)),
            pl.BlockSpec((KT, cs, cout2), lambda b, t: (0, 0, 0)),
            pl.BlockSpec((1, 1, n, cout2 // 2), lambda b, t: (0, 0, 0, 0)),
            pl.BlockSpec((1, 1, n, cout2 // 2), lambda b, t: (0, 0, 0, 0)),
        ],
        out_specs=pl.BlockSpec((1, 1, n, cout2 // 2), lambda b, t: (b, t, 0, 0)),
        out_shape=jax.ShapeDtypeStruct((B, T2, n, cout2 // 2), jnp.float32),
        compiler_params=pltpu.CompilerParams(
            dimension_semantics=("parallel", "arbitrary")),
    )(h2, h2, h2, W2, gamma, beta)
    return out


# speedup vs baseline: 4.0554x; 2.4411x over previous
import jax
import jax.numpy as jnp
from jax.experimental import pallas as pl
from jax.experimental.pallas import tpu as pltpu
from jax.scipy.linalg import block_diag

K = 3
R = 2
N = 512
KT = 3


def _dot(a, b, prec=jax.lax.Precision.DEFAULT):
    return jax.lax.dot_general(
        a, b, (((1,), (0,)), ((), ())),
        precision=prec, preferred_element_type=jnp.float32)


def _attn_stage_kernel(x0_ref, x1_ref, x2_ref, sup_ref, att_ref, w1_ref,
                       wtc_ref, wlbd_ref, wrbd_ref, out_ref, mscr):
    first = jnp.logical_and(pl.program_id(0) == 0, pl.program_id(1) == 0)

    @pl.when(first)
    def _():
        m0 = (att_ref[0] != 0).astype(jnp.float32)
        m1 = (att_ref[1] != 0).astype(jnp.float32)
        mscr[0] = m0
        mscr[1] = m1
        mscr[5] = m0 + m1
        for k in range(K):
            mscr[2 + k] = (sup_ref[k] != 0).astype(jnp.float32)

    xc = jnp.concatenate([x0_ref[0, 0], x1_ref[0, 0], x2_ref[0, 0]], axis=-1)
    y = _dot(xc, w1_ref[...])
    ch = y.shape[-1] // 2
    h = y[:, :ch] * jax.nn.sigmoid(y[:, ch:])

    wxa = _dot(h, wtc_ref[...])
    al = _dot(wxa, wlbd_ref[...])
    ar = jax.lax.dot_general(
        wrbd_ref[...], wxa, (((0,), (1,)), ((), ())),
        preferred_element_type=jnp.float32)

    cs = wxa.shape[-1] // K
    m0 = mscr[0]
    m1 = mscr[1]
    m01 = mscr[5]
    ps = []
    wxs = []
    for k in range(K):
        mk = mscr[2 + k]
        c = (R + 1) * k
        s = m0 * (al[:, c:c + 1] + ar[c:c + 1, :])
        s = s + m1 * (al[:, c + 1:c + 2] + ar[c + 1:c + 2, :])
        s = s + mk * (al[:, c + 2:c + 3] + ar[c + 2:c + 3, :])
        e = jnp.exp(jnp.where(m01 + mk > 0, s, -jnp.inf))
        z = jnp.sum(e, axis=-1, keepdims=True)
        p = e * jnp.where(z > 0, 1.0 / z, 0.0)
        ps.append(p)
        wxs.append(wxa[:, cs * k:cs * (k + 1)])
    pcat = jnp.concatenate(ps, axis=1)
    wxcat = jnp.concatenate(wxs, axis=0)
    out = _dot(pcat, wxcat)
    out_ref[0, 0] = jnp.where(out > 0, out, jnp.exp(jnp.minimum(out, 0.0)) - 1.0)


def _conv2_ln_kernel(h0_ref, h1_ref, h2_ref, w2_ref, g_ref, b_ref, out_ref):
    hc = jnp.concatenate([h0_ref[0, 0], h1_ref[0, 0], h2_ref[0, 0]], axis=-1)
    y = _dot(hc, w2_ref[...])
    ch = y.shape[-1] // 2
    g = y[:, :ch] * jax.nn.sigmoid(y[:, ch:])
    mu = jnp.mean(g)
    var = jnp.mean((g - mu) * (g - mu))
    out_ref[0, 0] = ((g - mu) / jnp.sqrt(var + 1e-6)) * g_ref[0, 0] + b_ref[0, 0]


def kernel(x, supports, atten_supports, W1, W_transform, W_left, W_right,
           W2, gamma, beta):
    B, T, n, cin = x.shape
    ch2 = W1.shape[-1]
    ch = ch2 // 2
    cs = W_transform.shape[-1]
    cout2 = W2.shape[-1]
    T1 = T - KT + 1
    T2 = T1 - KT + 1

    w1f = W1.reshape(KT * cin, ch2)
    wtc = jnp.moveaxis(W_transform, 0, 1).reshape(ch, K * cs)
    wlbd = block_diag(*[W_left[k].T for k in range(K)])
    wrbd = block_diag(*[W_right[k].T for k in range(K)])
    w2f = W2.reshape(KT * cs, cout2)

    h2 = pl.pallas_call(
        _attn_stage_kernel,
        grid=(B, T1),
        in_specs=[
            pl.BlockSpec((1, 1, n, cin), lambda b, t: (b, t, 0, 0)),
            pl.BlockSpec((1, 1, n, cin), lambda b, t: (b, t + 1, 0, 0)),
            pl.BlockSpec((1, 1, n, cin), lambda b, t: (b, t + 2, 0, 0)),
            pl.BlockSpec((K, n, n), lambda b, t: (0, 0, 0)),
            pl.BlockSpec((R, n, n), lambda b, t: (0, 0, 0)),
            pl.BlockSpec((KT * cin, ch2), lambda b, t: (0, 0)),
            pl.BlockSpec((ch, K * cs), lambda b, t: (0, 0)),
            pl.BlockSpec((K * cs, K * (R + 1)), lambda b, t: (0, 0)),
            pl.BlockSpec((K * cs, K * (R + 1)), lambda b, t: (0, 0)),
        ],
        out_specs=pl.BlockSpec((1, 1, n, cs), lambda b, t: (b, t, 0, 0)),
        out_shape=jax.ShapeDtypeStruct((B, T1, n, cs), jnp.float32),
        scratch_shapes=[pltpu.VMEM((2 * K, n, n), jnp.float32)],
        compiler_params=pltpu.CompilerParams(
            dimension_semantics=("arbitrary", "arbitrary")),
    )(x, x, x, supports, atten_supports, w1f, wtc, wlbd, wrbd)

    out = pl.pallas_call(
        _conv2_ln_kernel,
        grid=(B, T2),
        in_specs=[
            pl.BlockSpec((1, 1, n, cs), lambda b, t: (b, t, 0, 0)),
            pl.BlockSpec((1, 1, n, cs), lambda b, t: (b, t + 1, 0, 0)),
            pl.BlockSpec((1, 1, n, cs), lambda b, t: (b, t + 2, 0, 0)),
            pl.BlockSpec((KT * cs, cout2), lambda b, t: (0, 0)),
            pl.BlockSpec((1, 1, n, cout2 // 2), lambda b, t: (0, 0, 0, 0)),
            pl.BlockSpec((1, 1, n, cout2 // 2), lambda b, t: (0, 0, 0, 0)),
        ],
        out_specs=pl.BlockSpec((1, 1, n, cout2 // 2), lambda b, t: (b, t, 0, 0)),
        out_shape=jax.ShapeDtypeStruct((B, T2, n, cout2 // 2), jnp.float32),
        compiler_params=pltpu.CompilerParams(
            dimension_semantics=("parallel", "arbitrary")),
    )(h2, h2, h2, w2f, gamma, beta)
    return out


# speedup vs baseline: 5.0601x; 1.2477x over previous
import jax
import jax.numpy as jnp
from jax.experimental import pallas as pl
from jax.experimental.pallas import tpu as pltpu
from jax.scipy.linalg import block_diag

K = 3
R = 2
N = 512
KT = 3


def _dot(a, b):
    return jax.lax.dot_general(
        a, b, (((1,), (0,)), ((), ())),
        preferred_element_type=jnp.float32)


def _fused_kernel(x0_ref, x1_ref, x2_ref, sup_ref, att_ref, w1_ref,
                  wtc_ref, wlbd_ref, wrbd_ref, w2_ref, g_ref, bta_ref,
                  out_ref, mscr, hbuf):
    t = pl.program_id(1)
    first = jnp.logical_and(pl.program_id(0) == 0, t == 0)

    @pl.when(first)
    def _():
        m0 = (att_ref[0] != 0).astype(jnp.float32)
        m1 = (att_ref[1] != 0).astype(jnp.float32)
        mscr[0] = m0
        mscr[1] = m1
        for k in range(K):
            mk = (sup_ref[k] != 0).astype(jnp.float32)
            mscr[2 + k] = mk
            mscr[5 + k] = ((m0 + m1 + mk) > 0).astype(jnp.float32)

    xc = jnp.concatenate([x0_ref[0, 0], x1_ref[0, 0], x2_ref[0, 0]], axis=-1)
    y = _dot(xc, w1_ref[...])
    ch = y.shape[-1] // 2
    h = y[:, :ch] * jax.nn.sigmoid(y[:, ch:])

    wxa = _dot(h, wtc_ref[...])
    al = _dot(wxa, wlbd_ref[...])
    ar = jax.lax.dot_general(
        wrbd_ref[...], wxa, (((0,), (1,)), ((), ())),
        preferred_element_type=jnp.float32)

    cs = wxa.shape[-1] // K
    m0 = mscr[0]
    m1 = mscr[1]
    attn = jnp.zeros((N, cs), dtype=jnp.float32)
    for k in range(K):
        mk = mscr[2 + k]
        uk = mscr[5 + k]
        c = (R + 1) * k
        s = m0 * (al[:, c:c + 1] + ar[c:c + 1, :])
        s = s + m1 * (al[:, c + 1:c + 2] + ar[c + 1:c + 2, :])
        s = s + mk * (al[:, c + 2:c + 3] + ar[c + 2:c + 3, :])
        e = jnp.exp(s) * uk
        z = jnp.sum(e, axis=-1, keepdims=True)
        attn = attn + (1.0 / z) * _dot(e, wxa[:, cs * k:cs * (k + 1)])
    attn = jnp.where(attn > 0, attn, jnp.exp(jnp.minimum(attn, 0.0)) - 1.0)
    hbuf[t % 3] = attn

    @pl.when(t >= 2)
    def _():
        hc = jnp.concatenate(
            [hbuf[(t + 1) % 3], hbuf[(t + 2) % 3], hbuf[t % 3]], axis=-1)
        y2 = _dot(hc, w2_ref[...])
        co = y2.shape[-1] // 2
        g = y2[:, :co] * jax.nn.sigmoid(y2[:, co:])
        mu = jnp.mean(g)
        var = jnp.mean((g - mu) * (g - mu))
        out_ref[0, 0] = ((g - mu) / jnp.sqrt(var + 1e-6)) * g_ref[0, 0] \
            + bta_ref[0, 0]


def kernel(x, supports, atten_supports, W1, W_transform, W_left, W_right,
           W2, gamma, beta):
    B, T, n, cin = x.shape
    ch2 = W1.shape[-1]
    ch = ch2 // 2
    cs = W_transform.shape[-1]
    cout2 = W2.shape[-1]
    T1 = T - KT + 1
    T2 = T1 - KT + 1

    w1f = W1.reshape(KT * cin, ch2)
    wtc = jnp.moveaxis(W_transform, 0, 1).reshape(ch, K * cs)
    wlbd = block_diag(*[W_left[k].T for k in range(K)])
    wrbd = block_diag(*[W_right[k].T for k in range(K)])
    w2f = W2.reshape(KT * cs, cout2)

    out = pl.pallas_call(
        _fused_kernel,
        grid=(B, T1),
        in_specs=[
            pl.BlockSpec((1, 1, n, cin), lambda b, t: (b, t, 0, 0)),
            pl.BlockSpec((1, 1, n, cin), lambda b, t: (b, t + 1, 0, 0)),
            pl.BlockSpec((1, 1, n, cin), lambda b, t: (b, t + 2, 0, 0)),
            pl.BlockSpec((K, n, n), lambda b, t: (0, 0, 0)),
            pl.BlockSpec((R, n, n), lambda b, t: (0, 0, 0)),
            pl.BlockSpec((KT * cin, ch2), lambda b, t: (0, 0)),
            pl.BlockSpec((ch, K * cs), lambda b, t: (0, 0)),
            pl.BlockSpec((K * cs, K * (R + 1)), lambda b, t: (0, 0)),
            pl.BlockSpec((K * cs, K * (R + 1)), lambda b, t: (0, 0)),
            pl.BlockSpec((KT * cs, cout2), lambda b, t: (0, 0)),
            pl.BlockSpec((1, 1, n, cout2 // 2), lambda b, t: (0, 0, 0, 0)),
            pl.BlockSpec((1, 1, n, cout2 // 2), lambda b, t: (0, 0, 0, 0)),
        ],
        out_specs=pl.BlockSpec(
            (1, 1, n, cout2 // 2),
            lambda b, t: (b, jnp.maximum(t - 2, 0), 0, 0)),
        out_shape=jax.ShapeDtypeStruct((B, T2, n, cout2 // 2), jnp.float32),
        scratch_shapes=[
            pltpu.VMEM((5 + K, n, n), jnp.float32),
            pltpu.VMEM((3, n, cs), jnp.float32),
        ],
        compiler_params=pltpu.CompilerParams(
            dimension_semantics=("arbitrary", "arbitrary")),
    )(x, x, x, supports, atten_supports, w1f, wtc, wlbd, wrbd, w2f,
      gamma, beta)
    return out


# speedup vs baseline: 5.1687x; 1.0215x over previous
import jax
import jax.numpy as jnp
from jax.experimental import pallas as pl
from jax.experimental.pallas import tpu as pltpu
from jax.scipy.linalg import block_diag

K = 3
R = 2
N = 512
KT = 3


def _dot(a, b):
    return jax.lax.dot_general(
        a, b, (((1,), (0,)), ((), ())),
        preferred_element_type=jnp.float32)


def _fused_kernel(x0_ref, x1_ref, x2_ref, sup_ref, att_ref, w1_ref,
                  wtc_ref, wlbd_ref, wrbd_ref, w2_ref, g_ref, bta_ref,
                  out_ref, mscr, hbuf):
    t = pl.program_id(1)
    first = jnp.logical_and(pl.program_id(0) == 0, t == 0)

    @pl.when(first)
    def _():
        m0 = (att_ref[0] != 0).astype(jnp.float32)
        m1 = (att_ref[1] != 0).astype(jnp.float32)
        mscr[0] = m0
        mscr[1] = 1.0 - m0
        mscr[2] = m1
        mscr[3] = 1.0 - m1
        for k in range(K):
            mk = (sup_ref[k] != 0).astype(jnp.float32)
            uk = ((m0 + m1 + mk) > 0).astype(jnp.float32)
            mscr[4 + k] = mk
            mscr[7 + k] = (1.0 - mk) * uk

    xc = jnp.concatenate([x0_ref[0, 0], x1_ref[0, 0], x2_ref[0, 0]], axis=-1)
    y = _dot(xc, w1_ref[...])
    ch = y.shape[-1] // 2
    h = y[:, :ch] * jax.nn.sigmoid(y[:, ch:])

    wxa = _dot(h, wtc_ref[...])
    eal = jnp.exp(_dot(wxa, wlbd_ref[...]))
    ear = jnp.exp(jax.lax.dot_general(
        wrbd_ref[...], wxa, (((0,), (1,)), ((), ())),
        preferred_element_type=jnp.float32))

    cs = wxa.shape[-1] // K
    ones = jnp.ones((N, 1), dtype=jnp.float32)
    m0 = mscr[0]
    nm0 = mscr[1]
    m1 = mscr[2]
    nm1 = mscr[3]
    attn = jnp.zeros((N, cs), dtype=jnp.float32)
    for k in range(K):
        mk = mscr[4 + k]
        wk = mscr[7 + k]
        c = (R + 1) * k
        f = (m0 * eal[:, c:c + 1]) * ear[c:c + 1, :] + nm0
        f = f * ((m1 * eal[:, c + 1:c + 2]) * ear[c + 1:c + 2, :] + nm1)
        f = f * ((mk * eal[:, c + 2:c + 3]) * ear[c + 2:c + 3, :] + wk)
        aug = jnp.concatenate([wxa[:, cs * k:cs * (k + 1)], ones], axis=1)
        ew = _dot(f, aug)
        attn = attn + (1.0 / ew[:, cs:cs + 1]) * ew[:, :cs]
    attn = jnp.where(attn > 0, attn, jnp.exp(jnp.minimum(attn, 0.0)) - 1.0)
    hbuf[t % 3] = attn

    @pl.when(t >= 2)
    def _():
        hc = jnp.concatenate(
            [hbuf[(t + 1) % 3], hbuf[(t + 2) % 3], hbuf[t % 3]], axis=-1)
        y2 = _dot(hc, w2_ref[...])
        co = y2.shape[-1] // 2
        g = y2[:, :co] * jax.nn.sigmoid(y2[:, co:])
        mu = jnp.mean(g)
        var = jnp.mean((g - mu) * (g - mu))
        out_ref[0, 0] = ((g - mu) / jnp.sqrt(var + 1e-6)) * g_ref[0, 0] \
            + bta_ref[0, 0]


def kernel(x, supports, atten_supports, W1, W_transform, W_left, W_right,
           W2, gamma, beta):
    B, T, n, cin = x.shape
    ch2 = W1.shape[-1]
    ch = ch2 // 2
    cs = W_transform.shape[-1]
    cout2 = W2.shape[-1]
    T1 = T - KT + 1
    T2 = T1 - KT + 1

    w1f = W1.reshape(KT * cin, ch2)
    wtc = jnp.moveaxis(W_transform, 0, 1).reshape(ch, K * cs)
    wlbd = block_diag(*[W_left[k].T for k in range(K)])
    wrbd = block_diag(*[W_right[k].T for k in range(K)])
    w2f = W2.reshape(KT * cs, cout2)

    out = pl.pallas_call(
        _fused_kernel,
        grid=(B, T1),
        in_specs=[
            pl.BlockSpec((1, 1, n, cin), lambda b, t: (b, t, 0, 0)),
            pl.BlockSpec((1, 1, n, cin), lambda b, t: (b, t + 1, 0, 0)),
            pl.BlockSpec((1, 1, n, cin), lambda b, t: (b, t + 2, 0, 0)),
            pl.BlockSpec((K, n, n), lambda b, t: (0, 0, 0)),
            pl.BlockSpec((R, n, n), lambda b, t: (0, 0, 0)),
            pl.BlockSpec((KT * cin, ch2), lambda b, t: (0, 0)),
            pl.BlockSpec((ch, K * cs), lambda b, t: (0, 0)),
            pl.BlockSpec((K * cs, K * (R + 1)), lambda b, t: (0, 0)),
            pl.BlockSpec((K * cs, K * (R + 1)), lambda b, t: (0, 0)),
            pl.BlockSpec((KT * cs, cout2), lambda b, t: (0, 0)),
            pl.BlockSpec((1, 1, n, cout2 // 2), lambda b, t: (0, 0, 0, 0)),
            pl.BlockSpec((1, 1, n, cout2 // 2), lambda b, t: (0, 0, 0, 0)),
        ],
        out_specs=pl.BlockSpec(
            (1, 1, n, cout2 // 2),
            lambda b, t: (b, jnp.maximum(t - 2, 0), 0, 0)),
        out_shape=jax.ShapeDtypeStruct((B, T2, n, cout2 // 2), jnp.float32),
        scratch_shapes=[
            pltpu.VMEM((7 + K, n, n), jnp.float32),
            pltpu.VMEM((3, n, cs), jnp.float32),
        ],
        compiler_params=pltpu.CompilerParams(
            dimension_semantics=("arbitrary", "arbitrary")),
    )(x, x, x, supports, atten_supports, w1f, wtc, wlbd, wrbd, w2f,
      gamma, beta)
    return out
